# baseline (speedup 1.0000x reference)
import functools

import jax
import jax.numpy as jnp
from jax import lax
from jax.experimental import pallas as pl
from jax.experimental.pallas import tpu as pltpu
from jax.experimental.pallas import tpu_sc as plsc

N_NODES = 10000
NP = 10240
N_EDGES = 160000
FEAT = 256
HALF = 128
NS = 16
RPT = NP // NS
RSTAGE = 128

_mesh = plsc.VectorSubcoreMesh(core_axis_name="c", subcore_axis_name="s")


CHUNK2 = 128
NCH2 = 80
EPAD = NS * NCH2 * CHUNK2
LAST_REAL = (N_EDGES - (NS - 1) * NCH2 * CHUNK2) // CHUNK2


@functools.partial(
    pl.kernel,
    out_type=[jax.ShapeDtypeStruct((NP,), jnp.float32),
              jax.ShapeDtypeStruct((NP,), jnp.float32)],
    mesh=_mesh,
    scratch_types=[
        pltpu.VMEM((NCH2, CHUNK2), jnp.int32),
        pltpu.VMEM((CHUNK2,), jnp.float32),
        pltpu.VMEM((RPT,), jnp.float32),
        pltpu.VMEM_SHARED((NP,), jnp.float32),
        pltpu.SemaphoreType.DMA,
    ],
)
def _degrees(src3_hbm, dst3_hbm, osrc, odst, idx_v, ones_v, stage_v, acc_sh,
             sem):
    cid = lax.axis_index("c")
    sid = lax.axis_index("s")

    def _zrow(i, c):
        stage_v[pl.ds(i * 16, 16)] = jnp.zeros((16,), jnp.float32)
        return c
    lax.fori_loop(jnp.int32(0), jnp.int32(RPT // 16), _zrow, jnp.int32(0))
    for j in range(CHUNK2 // 16):
        ones_v[pl.ds(j * 16, 16)] = jnp.ones((16,), jnp.float32)

    rbase = pl.multiple_of(sid * RPT, 8)
    pltpu.sync_copy(stage_v, acc_sh.at[pl.ds(rbase, RPT)])

    @pl.when(cid == 0)
    def _():
        pltpu.sync_copy(src3_hbm.at[sid], idx_v)

    @pl.when(cid == 1)
    def _():
        pltpu.sync_copy(dst3_hbm.at[sid], idx_v)

    plsc.subcore_barrier()

    nch = jnp.where(sid == NS - 1, jnp.int32(LAST_REAL), jnp.int32(NCH2))

    def _body(j, c):
        pltpu.async_copy(ones_v, acc_sh.at[idx_v.at[j]], sem, add=True)
        return c
    lax.fori_loop(jnp.int32(0), nch, _body, jnp.int32(0))

    def _drain(j, c):
        pltpu.make_async_copy(
            ones_v, acc_sh.at[idx_v.at[jnp.int32(0)]], sem).wait()
        return c
    lax.fori_loop(jnp.int32(0), nch, _drain, jnp.int32(0))

    plsc.subcore_barrier()

    pltpu.sync_copy(acc_sh.at[pl.ds(rbase, RPT)], stage_v)

    @pl.when(cid == 0)
    def _():
        pltpu.sync_copy(stage_v, osrc.at[pl.ds(rbase, RPT)])

    @pl.when(cid == 1)
    def _():
        pltpu.sync_copy(stage_v, odst.at[pl.ds(rbase, RPT)])


CHUNKA = 64
NCHA = EPAD // NS // CHUNKA
NHA = NCHA // 4
NBUF = 4


@functools.partial(
    pl.kernel,
    out_type=[jax.ShapeDtypeStruct((NP, HALF), jnp.float32),
              jax.ShapeDtypeStruct((NP, HALF), jnp.float32)],
    mesh=_mesh,
    scratch_types=[
        pltpu.VMEM((NHA, CHUNKA), jnp.int32),
        pltpu.VMEM((NHA, CHUNKA), jnp.int32),
        [pltpu.VMEM((CHUNKA, HALF), jnp.float32)] * NBUF,
        [pltpu.SemaphoreType.DMA] * NBUF,
        [pltpu.SemaphoreType.DMA] * NBUF,
        pltpu.VMEM_SHARED((NP, HALF), jnp.float32),
    ],
)
def _edge_agg(xl_hbm, xr_hbm, src3_hbm, dst3_hbm, outl, outr,
              src_v, dst_v, rows, gsem, ssem, acc_sh):
    cid = lax.axis_index("c")
    sid = lax.axis_index("s")

    def _zrow(i, c):
        for j in range(HALF // 16):
            rows[0][i, pl.ds(j * 16, 16)] = jnp.zeros((16,), jnp.float32)
        return c
    lax.fori_loop(jnp.int32(0), jnp.int32(CHUNKA), _zrow, jnp.int32(0))
    for t in range(RPT // CHUNKA):
        start = pl.multiple_of(sid * RPT + t * CHUNKA, 8)
        pltpu.sync_copy(rows[0], acc_sh.at[pl.ds(start, CHUNKA)])

    plsc.subcore_barrier()

    def _run(x_hbm):
        for h in range(4):
            pltpu.sync_copy(src3_hbm.at[sid, pl.ds(h * NHA, NHA)], src_v)
            pltpu.sync_copy(dst3_hbm.at[sid, pl.ds(h * NHA, NHA)], dst_v)
            for b in range(NBUF - 1):
                pltpu.async_copy(
                    x_hbm.at[src_v.at[jnp.int32(b)]], rows[b], gsem[b])

            def _body(i, c):
                for b in range(NBUF):
                    j = NBUF * i + b
                    pltpu.make_async_copy(
                        x_hbm.at[src_v.at[j]], rows[b], gsem[b]).wait()
                    pltpu.async_copy(
                        rows[b], acc_sh.at[dst_v.at[j]], ssem[b], add=True)
                    b2 = (b + NBUF - 1) % NBUF

                    @pl.when(j + NBUF - 1 < NHA)
                    def _():
                        @pl.when(j >= 1)
                        def _():
                            pltpu.make_async_copy(
                                rows[b2], acc_sh.at[dst_v.at[j]],
                                ssem[b2]).wait()

                        pltpu.async_copy(
                            x_hbm.at[src_v.at[j + NBUF - 1]], rows[b2],
                            gsem[b2])
                return c
            lax.fori_loop(jnp.int32(0), jnp.int32(NHA // NBUF), _body,
                          jnp.int32(0))
            for b in range(NBUF):
                pltpu.make_async_copy(
                    rows[b], acc_sh.at[dst_v.at[jnp.int32(0)]],
                    ssem[b]).wait()

    @pl.when(cid == 0)
    def _():
        _run(xl_hbm)

    @pl.when(cid == 1)
    def _():
        _run(xr_hbm)

    plsc.subcore_barrier()

    for t in range(RPT // CHUNKA):
        start = pl.multiple_of(sid * RPT + t * CHUNKA, 8)
        sl = pl.ds(start, CHUNKA)
        pltpu.sync_copy(acc_sh.at[sl], rows[0])

        @pl.when(cid == 0)
        def _():
            pltpu.sync_copy(rows[0], outl.at[sl])

        @pl.when(cid == 1)
        def _():
            pltpu.sync_copy(rows[0], outr.at[sl])


_BM = 1000


def _norm(d):
    return jnp.where(d > 0.0, lax.rsqrt(d), 0.0)


def _mm1_body(h_ref, w_ref, ds_ref, ol_ref, or_ref):
    x = jnp.dot(h_ref[...], w_ref[...],
                preferred_element_type=jnp.float32,
                precision=lax.Precision.HIGHEST)
    x = x * _norm(ds_ref[...])
    ol_ref[...] = x[:, :HALF]
    or_ref[...] = x[:, HALF:]


def _mid_body(al_ref, ar_ref, ds_ref, dd_ref, b_ref, w_ref, ol_ref, or_ref):
    agg = jnp.concatenate([al_ref[...], ar_ref[...]], axis=1)
    t = jnp.maximum(agg * _norm(dd_ref[...]) + b_ref[...], 0.0)
    t = t * _norm(ds_ref[...])
    x = jnp.dot(t, w_ref[...],
                preferred_element_type=jnp.float32,
                precision=lax.Precision.HIGHEST)
    ol_ref[...] = x[:, :HALF]
    or_ref[...] = x[:, HALF:]


def _fin_body(al_ref, ar_ref, dd_ref, b_ref, o_ref):
    agg = jnp.concatenate([al_ref[...], ar_ref[...]], axis=1)
    o_ref[...] = jnp.maximum(agg * _norm(dd_ref[...]) + b_ref[...], 0.0)


_row_spec = pl.BlockSpec((_BM, FEAT), lambda i: (i, jnp.int32(0)))
_half_spec = pl.BlockSpec((_BM, HALF), lambda i: (i, jnp.int32(0)))
_deg_spec = pl.BlockSpec((_BM, 1), lambda i: (i, jnp.int32(0)))
_w_spec = pl.BlockSpec((FEAT, FEAT), lambda i: (jnp.int32(0), jnp.int32(0)))
_b_spec = pl.BlockSpec((1, FEAT), lambda i: (jnp.int32(0), jnp.int32(0)))
_grid = (N_NODES // _BM,)

_mm1 = pl.pallas_call(
    _mm1_body,
    grid=_grid,
    in_specs=[_row_spec, _w_spec, _deg_spec],
    out_specs=[_half_spec, _half_spec],
    out_shape=[jax.ShapeDtypeStruct((N_NODES, HALF), jnp.float32),
               jax.ShapeDtypeStruct((N_NODES, HALF), jnp.float32)],
)

_mid = pl.pallas_call(
    _mid_body,
    grid=_grid,
    in_specs=[_half_spec, _half_spec, _deg_spec, _deg_spec, _b_spec, _w_spec],
    out_specs=[_half_spec, _half_spec],
    out_shape=[jax.ShapeDtypeStruct((N_NODES, HALF), jnp.float32),
               jax.ShapeDtypeStruct((N_NODES, HALF), jnp.float32)],
)

_fin = pl.pallas_call(
    _fin_body,
    grid=_grid,
    in_specs=[_half_spec, _half_spec, _deg_spec, _b_spec],
    out_specs=_row_spec,
    out_shape=jax.ShapeDtypeStruct((N_NODES, FEAT), jnp.float32),
)


def kernel(h, edge_index, W1, b1, W2, b2):
    src = edge_index[0].astype(jnp.int32)
    dst = edge_index[1].astype(jnp.int32)
    h = h.astype(jnp.float32)

    fill = jnp.arange(EPAD - N_EDGES, dtype=jnp.int32)
    src_p = jnp.concatenate([src, fill % jnp.int32(N_NODES)])
    dst_p = jnp.concatenate(
        [dst, jnp.int32(N_NODES) + fill % jnp.int32(NP - N_NODES)])
    src3 = src_p.reshape(NS, NCH2, CHUNK2)
    dst3 = dst_p.reshape(NS, NCH2, CHUNK2)
    srca = src_p.reshape(NS, NCHA, CHUNKA)
    dsta = dst_p.reshape(NS, NCHA, CHUNKA)

    deg_src, deg_dst = _degrees(src3, dst3)
    ds2 = deg_src.reshape(NP, 1)
    dd2 = deg_dst.reshape(NP, 1)
    b1r = b1.astype(jnp.float32).reshape(1, FEAT)
    b2r = b2.astype(jnp.float32).reshape(1, FEAT)

    x1l, x1r = _mm1(h, W1.astype(jnp.float32), ds2)
    a1l, a1r = _edge_agg(x1l, x1r, srca, dsta)
    x2l, x2r = _mid(a1l, a1r, ds2, dd2, b1r, W2.astype(jnp.float32))
    a2l, a2r = _edge_agg(x2l, x2r, srca, dsta)
    return _fin(a2l, a2r, dd2, b2r)


# speedup vs baseline: 1.0261x; 1.0261x over previous
import functools

import jax
import jax.numpy as jnp
from jax import lax
from jax.experimental import pallas as pl
from jax.experimental.pallas import tpu as pltpu
from jax.experimental.pallas import tpu_sc as plsc

N_NODES = 10000
NP = 10240
N_EDGES = 160000
FEAT = 256
HALF = 128
NS = 16
RPT = NP // NS
RSTAGE = 128

_mesh = plsc.VectorSubcoreMesh(core_axis_name="c", subcore_axis_name="s")


CHUNK2 = 128
NCH2 = 80
EPAD = NS * NCH2 * CHUNK2
LAST_REAL = (N_EDGES - (NS - 1) * NCH2 * CHUNK2) // CHUNK2


@functools.partial(
    pl.kernel,
    out_type=[jax.ShapeDtypeStruct((NP,), jnp.float32),
              jax.ShapeDtypeStruct((NP,), jnp.float32)],
    mesh=_mesh,
    scratch_types=[
        pltpu.VMEM((NCH2, CHUNK2), jnp.int32),
        pltpu.VMEM((CHUNK2,), jnp.float32),
        pltpu.VMEM((RPT,), jnp.float32),
        pltpu.VMEM_SHARED((NP,), jnp.float32),
        pltpu.SemaphoreType.DMA,
    ],
)
def _degrees(src3_hbm, dst3_hbm, osrc, odst, idx_v, ones_v, stage_v, acc_sh,
             sem):
    cid = lax.axis_index("c")
    sid = lax.axis_index("s")

    def _zrow(i, c):
        stage_v[pl.ds(i * 16, 16)] = jnp.zeros((16,), jnp.float32)
        return c
    lax.fori_loop(jnp.int32(0), jnp.int32(RPT // 16), _zrow, jnp.int32(0))
    for j in range(CHUNK2 // 16):
        ones_v[pl.ds(j * 16, 16)] = jnp.ones((16,), jnp.float32)

    rbase = pl.multiple_of(sid * RPT, 8)
    pltpu.sync_copy(stage_v, acc_sh.at[pl.ds(rbase, RPT)])

    @pl.when(cid == 0)
    def _():
        pltpu.sync_copy(src3_hbm.at[sid], idx_v)

    @pl.when(cid == 1)
    def _():
        pltpu.sync_copy(dst3_hbm.at[sid], idx_v)

    plsc.subcore_barrier()

    nch = jnp.where(sid == NS - 1, jnp.int32(LAST_REAL), jnp.int32(NCH2))

    def _body(j, c):
        pltpu.async_copy(ones_v, acc_sh.at[idx_v.at[j]], sem, add=True)
        return c
    lax.fori_loop(jnp.int32(0), nch, _body, jnp.int32(0))

    def _drain(j, c):
        pltpu.make_async_copy(
            ones_v, acc_sh.at[idx_v.at[jnp.int32(0)]], sem).wait()
        return c
    lax.fori_loop(jnp.int32(0), nch, _drain, jnp.int32(0))

    plsc.subcore_barrier()

    pltpu.sync_copy(acc_sh.at[pl.ds(rbase, RPT)], stage_v)

    @pl.when(cid == 0)
    def _():
        pltpu.sync_copy(stage_v, osrc.at[pl.ds(rbase, RPT)])

    @pl.when(cid == 1)
    def _():
        pltpu.sync_copy(stage_v, odst.at[pl.ds(rbase, RPT)])


NHA = NCH2 // 2


@functools.partial(
    pl.kernel,
    out_type=[jax.ShapeDtypeStruct((NP, HALF), jnp.float32),
              jax.ShapeDtypeStruct((NP, HALF), jnp.float32)],
    mesh=_mesh,
    scratch_types=[
        pltpu.VMEM((NHA, CHUNK2), jnp.int32),
        pltpu.VMEM((NHA, CHUNK2), jnp.int32),
        pltpu.VMEM((CHUNK2, HALF), jnp.float32),
        pltpu.VMEM((CHUNK2, HALF), jnp.float32),
        pltpu.VMEM_SHARED((NP, HALF), jnp.float32),
        pltpu.SemaphoreType.DMA,
        pltpu.SemaphoreType.DMA,
    ],
)
def _edge_agg(xl_hbm, xr_hbm, src3_hbm, dst3_hbm, outl, outr,
              src_v, dst_v, rows0_v, rows1_v, acc_sh, sem0, sem1):
    cid = lax.axis_index("c")
    sid = lax.axis_index("s")

    def _zrow(i, c):
        for j in range(HALF // 16):
            rows0_v[i, pl.ds(j * 16, 16)] = jnp.zeros((16,), jnp.float32)
        return c
    lax.fori_loop(jnp.int32(0), jnp.int32(CHUNK2), _zrow, jnp.int32(0))
    for t in range(RPT // CHUNK2):
        start = pl.multiple_of(sid * RPT + t * CHUNK2, 8)
        pltpu.sync_copy(rows0_v, acc_sh.at[pl.ds(start, CHUNK2)])

    plsc.subcore_barrier()

    def _run(x_hbm):
        for h in range(2):
            pltpu.sync_copy(src3_hbm.at[sid, pl.ds(h * NHA, NHA)], src_v)
            pltpu.sync_copy(dst3_hbm.at[sid, pl.ds(h * NHA, NHA)], dst_v)
            pltpu.async_copy(x_hbm.at[src_v.at[jnp.int32(0)]], rows0_v, sem0)

            def _body(i, c):
                j0 = 2 * i
                j1 = j0 + 1
                pltpu.async_copy(x_hbm.at[src_v.at[j1]], rows1_v, sem1)
                pltpu.make_async_copy(
                    x_hbm.at[src_v.at[j0]], rows0_v, sem0).wait()
                pltpu.sync_copy(rows0_v, acc_sh.at[dst_v.at[j0]], add=True)

                @pl.when(j0 + 2 < NHA)
                def _():
                    pltpu.async_copy(x_hbm.at[src_v.at[j0 + 2]], rows0_v, sem0)

                pltpu.make_async_copy(
                    x_hbm.at[src_v.at[j1]], rows1_v, sem1).wait()
                pltpu.sync_copy(rows1_v, acc_sh.at[dst_v.at[j1]], add=True)
                return c
            lax.fori_loop(jnp.int32(0), jnp.int32(NHA // 2), _body,
                          jnp.int32(0))

    @pl.when(cid == 0)
    def _():
        _run(xl_hbm)

    @pl.when(cid == 1)
    def _():
        _run(xr_hbm)

    plsc.subcore_barrier()

    for t in range(RPT // CHUNK2):
        start = pl.multiple_of(sid * RPT + t * CHUNK2, 8)
        sl = pl.ds(start, CHUNK2)
        pltpu.sync_copy(acc_sh.at[sl], rows0_v)

        @pl.when(cid == 0)
        def _():
            pltpu.sync_copy(rows0_v, outl.at[sl])

        @pl.when(cid == 1)
        def _():
            pltpu.sync_copy(rows0_v, outr.at[sl])


_BM = 1000


def _norm(d):
    return jnp.where(d > 0.0, lax.rsqrt(d), 0.0)


def _mm1_body(h_ref, w_ref, ds_ref, ol_ref, or_ref):
    x = jnp.dot(h_ref[...], w_ref[...],
                preferred_element_type=jnp.float32,
                precision=lax.Precision.HIGHEST)
    x = x * _norm(ds_ref[...])
    ol_ref[...] = x[:, :HALF]
    or_ref[...] = x[:, HALF:]


def _mid_body(al_ref, ar_ref, ds_ref, dd_ref, b_ref, w_ref, ol_ref, or_ref):
    agg = jnp.concatenate([al_ref[...], ar_ref[...]], axis=1)
    t = jnp.maximum(agg * _norm(dd_ref[...]) + b_ref[...], 0.0)
    t = t * _norm(ds_ref[...])
    x = jnp.dot(t, w_ref[...],
                preferred_element_type=jnp.float32,
                precision=lax.Precision.HIGHEST)
    ol_ref[...] = x[:, :HALF]
    or_ref[...] = x[:, HALF:]


def _fin_body(al_ref, ar_ref, dd_ref, b_ref, o_ref):
    agg = jnp.concatenate([al_ref[...], ar_ref[...]], axis=1)
    o_ref[...] = jnp.maximum(agg * _norm(dd_ref[...]) + b_ref[...], 0.0)


_row_spec = pl.BlockSpec((_BM, FEAT), lambda i: (i, jnp.int32(0)))
_half_spec = pl.BlockSpec((_BM, HALF), lambda i: (i, jnp.int32(0)))
_deg_spec = pl.BlockSpec((_BM, 1), lambda i: (i, jnp.int32(0)))
_w_spec = pl.BlockSpec((FEAT, FEAT), lambda i: (jnp.int32(0), jnp.int32(0)))
_b_spec = pl.BlockSpec((1, FEAT), lambda i: (jnp.int32(0), jnp.int32(0)))
_grid = (N_NODES // _BM,)

_mm1 = pl.pallas_call(
    _mm1_body,
    grid=_grid,
    in_specs=[_row_spec, _w_spec, _deg_spec],
    out_specs=[_half_spec, _half_spec],
    out_shape=[jax.ShapeDtypeStruct((N_NODES, HALF), jnp.float32),
               jax.ShapeDtypeStruct((N_NODES, HALF), jnp.float32)],
)

_mid = pl.pallas_call(
    _mid_body,
    grid=_grid,
    in_specs=[_half_spec, _half_spec, _deg_spec, _deg_spec, _b_spec, _w_spec],
    out_specs=[_half_spec, _half_spec],
    out_shape=[jax.ShapeDtypeStruct((N_NODES, HALF), jnp.float32),
               jax.ShapeDtypeStruct((N_NODES, HALF), jnp.float32)],
)

_fin = pl.pallas_call(
    _fin_body,
    grid=_grid,
    in_specs=[_half_spec, _half_spec, _deg_spec, _b_spec],
    out_specs=_row_spec,
    out_shape=jax.ShapeDtypeStruct((N_NODES, FEAT), jnp.float32),
)


def kernel(h, edge_index, W1, b1, W2, b2):
    src = edge_index[0].astype(jnp.int32)
    dst = edge_index[1].astype(jnp.int32)
    h = h.astype(jnp.float32)

    fill = jnp.arange(EPAD - N_EDGES, dtype=jnp.int32)
    src_p = jnp.concatenate([src, fill % jnp.int32(N_NODES)])
    dst_p = jnp.concatenate(
        [dst, jnp.int32(N_NODES) + fill % jnp.int32(NP - N_NODES)])
    src3 = src_p.reshape(NS, NCH2, CHUNK2)
    dst3 = dst_p.reshape(NS, NCH2, CHUNK2)

    deg_src, deg_dst = _degrees(src3, dst3)
    ds2 = deg_src.reshape(NP, 1)
    dd2 = deg_dst.reshape(NP, 1)
    b1r = b1.astype(jnp.float32).reshape(1, FEAT)
    b2r = b2.astype(jnp.float32).reshape(1, FEAT)

    x1l, x1r = _mm1(h, W1.astype(jnp.float32), ds2)
    a1l, a1r = _edge_agg(x1l, x1r, src3, dst3)
    x2l, x2r = _mid(a1l, a1r, ds2, dd2, b1r, W2.astype(jnp.float32))
    a2l, a2r = _edge_agg(x2l, x2r, src3, dst3)
    return _fin(a2l, a2r, dd2, b2r)


# speedup vs baseline: 1.0744x; 1.0471x over previous
import functools

import jax
import jax.numpy as jnp
from jax import lax
from jax.experimental import pallas as pl
from jax.experimental.pallas import tpu as pltpu
from jax.experimental.pallas import tpu_sc as plsc

N_NODES = 10000
NP = 10240
N_EDGES = 160000
FEAT = 256
HALF = 128
NS = 16
RPT = NP // NS
RSTAGE = 128

_mesh = plsc.VectorSubcoreMesh(core_axis_name="c", subcore_axis_name="s")


CHUNK2 = 128
NCH2 = 80
EPAD = NS * NCH2 * CHUNK2
LAST_REAL = (N_EDGES - (NS - 1) * NCH2 * CHUNK2) // CHUNK2


@functools.partial(
    pl.kernel,
    out_type=[jax.ShapeDtypeStruct((NP,), jnp.float32),
              jax.ShapeDtypeStruct((NP,), jnp.float32)],
    mesh=_mesh,
    scratch_types=[
        pltpu.VMEM((NCH2, CHUNK2), jnp.int32),
        pltpu.VMEM((CHUNK2,), jnp.float32),
        pltpu.VMEM((RPT,), jnp.float32),
        pltpu.VMEM_SHARED((NP,), jnp.float32),
        pltpu.SemaphoreType.DMA,
    ],
)
def _degrees(src3_hbm, dst3_hbm, osrc, odst, idx_v, ones_v, stage_v, acc_sh,
             sem):
    cid = lax.axis_index("c")
    sid = lax.axis_index("s")

    def _zrow(i, c):
        stage_v[pl.ds(i * 16, 16)] = jnp.zeros((16,), jnp.float32)
        return c
    lax.fori_loop(jnp.int32(0), jnp.int32(RPT // 16), _zrow, jnp.int32(0))
    for j in range(CHUNK2 // 16):
        ones_v[pl.ds(j * 16, 16)] = jnp.ones((16,), jnp.float32)

    rbase = pl.multiple_of(sid * RPT, 8)
    pltpu.sync_copy(stage_v, acc_sh.at[pl.ds(rbase, RPT)])

    @pl.when(cid == 0)
    def _():
        pltpu.sync_copy(src3_hbm.at[sid], idx_v)

    @pl.when(cid == 1)
    def _():
        pltpu.sync_copy(dst3_hbm.at[sid], idx_v)

    plsc.subcore_barrier()

    nch = jnp.where(sid == NS - 1, jnp.int32(LAST_REAL), jnp.int32(NCH2))

    def _body(j, c):
        pltpu.async_copy(ones_v, acc_sh.at[idx_v.at[j]], sem, add=True)
        return c
    lax.fori_loop(jnp.int32(0), nch, _body, jnp.int32(0))

    def _drain(j, c):
        pltpu.make_async_copy(
            ones_v, acc_sh.at[idx_v.at[jnp.int32(0)]], sem).wait()
        return c
    lax.fori_loop(jnp.int32(0), nch, _drain, jnp.int32(0))

    plsc.subcore_barrier()

    pltpu.sync_copy(acc_sh.at[pl.ds(rbase, RPT)], stage_v)

    @pl.when(cid == 0)
    def _():
        pltpu.sync_copy(stage_v, osrc.at[pl.ds(rbase, RPT)])

    @pl.when(cid == 1)
    def _():
        pltpu.sync_copy(stage_v, odst.at[pl.ds(rbase, RPT)])


NHA = NCH2 // 2


@functools.partial(
    pl.kernel,
    out_type=[jax.ShapeDtypeStruct((NP, HALF), jnp.float32),
              jax.ShapeDtypeStruct((NP, HALF), jnp.float32)],
    mesh=_mesh,
    scratch_types=[
        pltpu.VMEM((NHA, CHUNK2), jnp.int32),
        pltpu.VMEM((NHA, CHUNK2), jnp.int32),
        pltpu.VMEM((CHUNK2, HALF), jnp.float32),
        pltpu.VMEM((CHUNK2, HALF), jnp.float32),
        pltpu.VMEM_SHARED((NP, HALF), jnp.float32),
        pltpu.SemaphoreType.DMA,
        pltpu.SemaphoreType.DMA,
    ],
)
def _edge_agg(xl_hbm, xr_hbm, src3_hbm, dst3_hbm, outl, outr,
              src_v, dst_v, rows0_v, rows1_v, acc_sh, sem0, sem1):
    cid = lax.axis_index("c")
    sid = lax.axis_index("s")

    def _zrow(i, c):
        for j in range(HALF // 16):
            rows0_v[i, pl.ds(j * 16, 16)] = jnp.zeros((16,), jnp.float32)
        return c
    lax.fori_loop(jnp.int32(0), jnp.int32(CHUNK2), _zrow, jnp.int32(0))
    for t in range(RPT // CHUNK2):
        start = pl.multiple_of(sid * RPT + t * CHUNK2, 8)
        pltpu.sync_copy(rows0_v, acc_sh.at[pl.ds(start, CHUNK2)])

    plsc.subcore_barrier()

    def _run(x_hbm):
        for h in range(2):
            pltpu.sync_copy(src3_hbm.at[sid, pl.ds(h * NHA, NHA)], src_v)
            pltpu.sync_copy(dst3_hbm.at[sid, pl.ds(h * NHA, NHA)], dst_v)
            pltpu.async_copy(x_hbm.at[src_v.at[jnp.int32(0)]], rows0_v, sem0)

            def _body(i, c):
                j0 = 2 * i
                j1 = j0 + 1
                pltpu.async_copy(x_hbm.at[src_v.at[j1]], rows1_v, sem1)
                pltpu.make_async_copy(
                    x_hbm.at[src_v.at[j0]], rows0_v, sem0).wait()
                pltpu.sync_copy(rows0_v, acc_sh.at[dst_v.at[j0]], add=True)

                @pl.when(j0 + 2 < NHA)
                def _():
                    pltpu.async_copy(x_hbm.at[src_v.at[j0 + 2]], rows0_v, sem0)

                pltpu.make_async_copy(
                    x_hbm.at[src_v.at[j1]], rows1_v, sem1).wait()
                pltpu.sync_copy(rows1_v, acc_sh.at[dst_v.at[j1]], add=True)
                return c
            lax.fori_loop(jnp.int32(0), jnp.int32(NHA // 2), _body,
                          jnp.int32(0))

    @pl.when(cid == 0)
    def _():
        _run(xl_hbm)

    @pl.when(cid == 1)
    def _():
        _run(xr_hbm)

    plsc.subcore_barrier()

    for t in range(RPT // CHUNK2):
        start = pl.multiple_of(sid * RPT + t * CHUNK2, 8)
        sl = pl.ds(start, CHUNK2)
        pltpu.sync_copy(acc_sh.at[sl], rows0_v)

        @pl.when(cid == 0)
        def _():
            pltpu.sync_copy(rows0_v, outl.at[sl])

        @pl.when(cid == 1)
        def _():
            pltpu.sync_copy(rows0_v, outr.at[sl])


_BM = 2000


def _norm(d):
    return jnp.where(d > 0.0, lax.rsqrt(d), 0.0)


def _mm1_body(h_ref, w_ref, ds_ref, ol_ref, or_ref):
    x = jnp.dot(h_ref[...], w_ref[...],
                preferred_element_type=jnp.float32,
                precision=lax.Precision.DEFAULT)
    x = x * _norm(ds_ref[...])
    ol_ref[...] = x[:, :HALF]
    or_ref[...] = x[:, HALF:]


def _mid_body(al_ref, ar_ref, ds_ref, dd_ref, b_ref, w_ref, ol_ref, or_ref):
    agg = jnp.concatenate([al_ref[...], ar_ref[...]], axis=1)
    t = jnp.maximum(agg * _norm(dd_ref[...]) + b_ref[...], 0.0)
    t = t * _norm(ds_ref[...])
    x = jnp.dot(t, w_ref[...],
                preferred_element_type=jnp.float32,
                precision=lax.Precision.DEFAULT)
    ol_ref[...] = x[:, :HALF]
    or_ref[...] = x[:, HALF:]


def _fin_body(al_ref, ar_ref, dd_ref, b_ref, o_ref):
    agg = jnp.concatenate([al_ref[...], ar_ref[...]], axis=1)
    o_ref[...] = jnp.maximum(agg * _norm(dd_ref[...]) + b_ref[...], 0.0)


_row_spec = pl.BlockSpec((_BM, FEAT), lambda i: (i, jnp.int32(0)))
_half_spec = pl.BlockSpec((_BM, HALF), lambda i: (i, jnp.int32(0)))
_deg_spec = pl.BlockSpec((_BM, 1), lambda i: (i, jnp.int32(0)))
_w_spec = pl.BlockSpec((FEAT, FEAT), lambda i: (jnp.int32(0), jnp.int32(0)))
_b_spec = pl.BlockSpec((1, FEAT), lambda i: (jnp.int32(0), jnp.int32(0)))
_grid = (N_NODES // _BM,)

_mm1 = pl.pallas_call(
    _mm1_body,
    grid=_grid,
    in_specs=[_row_spec, _w_spec, _deg_spec],
    out_specs=[_half_spec, _half_spec],
    out_shape=[jax.ShapeDtypeStruct((N_NODES, HALF), jnp.float32),
               jax.ShapeDtypeStruct((N_NODES, HALF), jnp.float32)],
)

_mid = pl.pallas_call(
    _mid_body,
    grid=_grid,
    in_specs=[_half_spec, _half_spec, _deg_spec, _deg_spec, _b_spec, _w_spec],
    out_specs=[_half_spec, _half_spec],
    out_shape=[jax.ShapeDtypeStruct((N_NODES, HALF), jnp.float32),
               jax.ShapeDtypeStruct((N_NODES, HALF), jnp.float32)],
)

_fin = pl.pallas_call(
    _fin_body,
    grid=_grid,
    in_specs=[_half_spec, _half_spec, _deg_spec, _b_spec],
    out_specs=_row_spec,
    out_shape=jax.ShapeDtypeStruct((N_NODES, FEAT), jnp.float32),
)


def kernel(h, edge_index, W1, b1, W2, b2):
    src = edge_index[0].astype(jnp.int32)
    dst = edge_index[1].astype(jnp.int32)
    h = h.astype(jnp.float32)

    fill = jnp.arange(EPAD - N_EDGES, dtype=jnp.int32)
    src_p = jnp.concatenate([src, fill % jnp.int32(N_NODES)])
    dst_p = jnp.concatenate(
        [dst, jnp.int32(N_NODES) + fill % jnp.int32(NP - N_NODES)])
    src3 = src_p.reshape(NS, NCH2, CHUNK2)
    dst3 = dst_p.reshape(NS, NCH2, CHUNK2)

    deg_src, deg_dst = _degrees(src3, dst3)
    ds2 = deg_src.reshape(NP, 1)
    dd2 = deg_dst.reshape(NP, 1)
    b1r = b1.astype(jnp.float32).reshape(1, FEAT)
    b2r = b2.astype(jnp.float32).reshape(1, FEAT)

    x1l, x1r = _mm1(h, W1.astype(jnp.float32), ds2)
    a1l, a1r = _edge_agg(x1l, x1r, src3, dst3)
    x2l, x2r = _mid(a1l, a1r, ds2, dd2, b1r, W2.astype(jnp.float32))
    a2l, a2r = _edge_agg(x2l, x2r, src3, dst3)
    return _fin(a2l, a2r, dd2, b2r)
